# baseline (speedup 1.0000x reference)
import math

import jax
import jax.numpy as jnp
from jax.experimental import pallas as pl

N_HEADS = 8
CH = 64
T = 4096
BT = 512
NSUB = 2
SUB = BT // NSUB


def _attn_kernel(ne_ref, wp_ref, q_ref, k_ref, v_ref, out_ref):
    bias = jax.lax.dot_general(
        wp_ref[0], ne_ref[...], (((1,), (1,)), ((), ())),
        preferred_element_type=jnp.float32)
    scale2 = math.log2(math.e) / math.sqrt(CH)
    qb_all = ((q_ref[0] + bias[0:CH]) * scale2).astype(jnp.bfloat16)
    kb = (k_ref[0] + bias[CH:2 * CH]).astype(jnp.bfloat16)
    vb = v_ref[0] + bias[2 * CH:3 * CH]
    vb1 = jnp.concatenate([vb, jnp.ones((1, T), jnp.float32)], axis=0)
    for j in range(NSUB):
        qb = qb_all[:, j * SUB:(j + 1) * SUB]
        w = jax.lax.dot_general(qb, kb, (((0,), (0,)), ((), ())),
                                preferred_element_type=jnp.float32)
        w = w - jnp.max(w, axis=1, keepdims=True)
        e = jnp.exp2(w)
        acc = jax.lax.dot_general(vb1, e, (((1,), (1,)), ((), ())),
                                  preferred_element_type=jnp.float32)
        out_ref[0, :, j * SUB:(j + 1) * SUB] = acc[0:CH] * (1.0 / acc[CH:CH + 1])


def kernel(qkv, null_emb, W_prompt):
    bs, width, length = qkv.shape
    emb = null_emb.shape[0]
    qkv_r = qkv.reshape(N_HEADS, 3 * CH, length)
    ne = null_emb.reshape(1, emb)
    wp = W_prompt.reshape(N_HEADS, 3 * CH, emb)
    out = pl.pallas_call(
        _attn_kernel,
        grid=(N_HEADS, T // BT),
        in_specs=[
            pl.BlockSpec((1, emb), lambda h, t: (0, 0)),
            pl.BlockSpec((1, 3 * CH, emb), lambda h, t: (h, 0, 0)),
            pl.BlockSpec((1, CH, BT), lambda h, t: (h, 0, t)),
            pl.BlockSpec((1, CH, T), lambda h, t: (h, 1, 0)),
            pl.BlockSpec((1, CH, T), lambda h, t: (h, 2, 0)),
        ],
        out_specs=pl.BlockSpec((1, CH, BT), lambda h, t: (h, 0, t)),
        out_shape=jax.ShapeDtypeStruct((N_HEADS, CH, T), jnp.float32),
    )(ne, wp, qkv_r, qkv_r, qkv_r)
    return out.reshape(bs, N_HEADS * CH, length)


# speedup vs baseline: 1.5615x; 1.5615x over previous
import math

import jax
import jax.numpy as jnp
from jax.experimental import pallas as pl

N_HEADS = 8
CH = 64
T = 4096
BT = 1024


def _attn_kernel(ne_ref, wp_ref, q_ref, k_ref, v_ref, out_ref):
    bias = jax.lax.dot_general(
        wp_ref[0], ne_ref[...], (((1,), (1,)), ((), ())),
        preferred_element_type=jnp.float32)
    scale2 = math.log2(math.e) / math.sqrt(CH)
    qb_all = ((q_ref[0] + bias[0:CH]) * scale2).astype(jnp.bfloat16)
    kb = (k_ref[0] + bias[CH:2 * CH]).astype(jnp.bfloat16)
    vb = v_ref[0] + bias[2 * CH:3 * CH]
    vb1 = jnp.concatenate([vb, jnp.ones((1, T), jnp.float32)], axis=0)
    w = jax.lax.dot_general(qb_all, kb, (((0,), (0,)), ((), ())),
                            preferred_element_type=jnp.float32)
    w = w - jnp.max(w, axis=1, keepdims=True)
    e = jnp.exp2(w)
    acc = jax.lax.dot_general(vb1, e, (((1,), (1,)), ((), ())),
                              preferred_element_type=jnp.float32)
    out_ref[0] = acc[0:CH] * (1.0 / acc[CH:CH + 1])


def kernel(qkv, null_emb, W_prompt):
    bs, width, length = qkv.shape
    emb = null_emb.shape[0]
    qkv_r = qkv.reshape(N_HEADS, 3 * CH, length)
    ne = null_emb.reshape(1, emb)
    wp = W_prompt.reshape(N_HEADS, 3 * CH, emb)
    out = pl.pallas_call(
        _attn_kernel,
        grid=(N_HEADS, T // BT),
        in_specs=[
            pl.BlockSpec((1, emb), lambda h, t: (0, 0)),
            pl.BlockSpec((1, 3 * CH, emb), lambda h, t: (h, 0, 0)),
            pl.BlockSpec((1, CH, BT), lambda h, t: (h, 0, t)),
            pl.BlockSpec((1, CH, T), lambda h, t: (h, 1, 0)),
            pl.BlockSpec((1, CH, T), lambda h, t: (h, 2, 0)),
        ],
        out_specs=pl.BlockSpec((1, CH, BT), lambda h, t: (h, 0, t)),
        out_shape=jax.ShapeDtypeStruct((N_HEADS, CH, T), jnp.float32),
    )(ne, wp, qkv_r, qkv_r, qkv_r)
    return out.reshape(bs, N_HEADS * CH, length)


# speedup vs baseline: 1.6025x; 1.0263x over previous
import math

import jax
import jax.numpy as jnp
from jax.experimental import pallas as pl

N_HEADS = 8
CH = 64
T = 4096
BT = 2048


def _attn_kernel(ne_ref, wp_ref, q_ref, k_ref, v_ref, out_ref):
    bias = jax.lax.dot_general(
        wp_ref[0], ne_ref[...], (((1,), (1,)), ((), ())),
        preferred_element_type=jnp.float32)
    scale2 = math.log2(math.e) / math.sqrt(CH)
    qb_all = ((q_ref[0] + bias[0:CH]) * scale2).astype(jnp.bfloat16)
    kb = (k_ref[0] + bias[CH:2 * CH]).astype(jnp.bfloat16)
    vb = v_ref[0] + bias[2 * CH:3 * CH]
    vb1 = jnp.concatenate([vb, jnp.ones((1, T), jnp.float32)], axis=0)
    w = jax.lax.dot_general(qb_all, kb, (((0,), (0,)), ((), ())),
                            preferred_element_type=jnp.float32)
    w = w - jnp.max(w, axis=1, keepdims=True)
    e = jnp.exp2(w)
    acc = jax.lax.dot_general(vb1, e, (((1,), (1,)), ((), ())),
                              preferred_element_type=jnp.float32)
    out_ref[0] = acc[0:CH] * (1.0 / acc[CH:CH + 1])


def kernel(qkv, null_emb, W_prompt):
    bs, width, length = qkv.shape
    emb = null_emb.shape[0]
    qkv_r = qkv.reshape(N_HEADS, 3 * CH, length)
    ne = null_emb.reshape(1, emb)
    wp = W_prompt.reshape(N_HEADS, 3 * CH, emb)
    out = pl.pallas_call(
        _attn_kernel,
        grid=(N_HEADS, T // BT),
        in_specs=[
            pl.BlockSpec((1, emb), lambda h, t: (0, 0)),
            pl.BlockSpec((1, 3 * CH, emb), lambda h, t: (h, 0, 0)),
            pl.BlockSpec((1, CH, BT), lambda h, t: (h, 0, t)),
            pl.BlockSpec((1, CH, T), lambda h, t: (h, 1, 0)),
            pl.BlockSpec((1, CH, T), lambda h, t: (h, 2, 0)),
        ],
        out_specs=pl.BlockSpec((1, CH, BT), lambda h, t: (h, 0, t)),
        out_shape=jax.ShapeDtypeStruct((N_HEADS, CH, T), jnp.float32),
    )(ne, wp, qkv_r, qkv_r, qkv_r)
    return out.reshape(bs, N_HEADS * CH, length)
